# baseline (speedup 1.0000x reference)
import functools

import jax
import jax.numpy as jnp
from jax import lax
from jax.experimental import pallas as pl
from jax.experimental.pallas import tpu as pltpu
from jax.experimental.pallas import tpu_sc as plsc

VOCAB = 100000
EMBED = 128
B = 16384
NNEG = 20
NCTX = NNEG + 1
NLANE = 16
NREG = EMBED // NLANE
SROW = 32

NC = 2
NS = 16
NW = NC * NS
RW = B // NW
C = 8
NCHUNK = RW // C
NI = C * NNEG
NSLOT = 5
NIX = 10
GDIST = NSLOT - 1
IDIST = 4

_DNUMS = lax.GatherDimensionNumbers(
    offset_dims=(), collapsed_slice_dims=(0,), start_index_map=(0,))


def _lane_rot(p, sh):
  perm = ((lax.iota(jnp.int32, NLANE) + sh) % NLANE)[:, None]
  return lax.gather(p, perm, _DNUMS, (1,),
                    mode=lax.GatherScatterMode.PROMISE_IN_BOUNDS)


def _allsum(p):
  for sh in (8, 4, 2, 1):
    p = p + _lane_rot(p, sh)
  return p


def _sc_scores(word_pos, ctx_pos, neg_pos, word_table, ctx_table):
  mesh = plsc.VectorSubcoreMesh(core_axis_name="c", subcore_axis_name="s")

  @functools.partial(
      pl.kernel,
      mesh=mesh,
      out_type=jax.ShapeDtypeStruct((B * SROW,), jnp.float32),
      scratch_types=[
          pltpu.VMEM((RW,), jnp.int32),
          pltpu.VMEM((RW,), jnp.int32),
          *[pltpu.VMEM((C, NNEG), jnp.int32) for _ in range(NIX)],
          *[pltpu.VMEM((C, EMBED), jnp.float32) for _ in range(NSLOT)],
          *[pltpu.VMEM((C, EMBED), jnp.float32) for _ in range(NSLOT)],
          *[pltpu.VMEM((NI, EMBED), jnp.float32) for _ in range(NSLOT)],
          *[pltpu.VMEM((C * SROW,), jnp.float32) for _ in range(NSLOT)],
          *[pltpu.SemaphoreType.DMA for _ in range(NIX)],
          *[pltpu.SemaphoreType.DMA for _ in range(NSLOT)],
          *[pltpu.SemaphoreType.DMA for _ in range(NSLOT)],
      ],
  )
  def k(wp_hbm, cp_hbm, np_hbm, wt_hbm, ct_hbm, out_hbm,
        widx_v, pidx_v, *rest):
    nix = rest[0:NIX]
    wrs = rest[NIX:NIX + NSLOT]
    prs = rest[NIX + NSLOT:NIX + 2 * NSLOT]
    nrs = rest[NIX + 2 * NSLOT:NIX + 3 * NSLOT]
    wbs = rest[NIX + 3 * NSLOT:NIX + 4 * NSLOT]
    isems = rest[NIX + 4 * NSLOT:NIX + 4 * NSLOT + NIX]
    sems = rest[NIX + 4 * NSLOT + NIX:NIX + 5 * NSLOT + NIX]
    wsems = rest[NIX + 5 * NSLOT + NIX:]
    wid = lax.axis_index("s") * NC + lax.axis_index("c")
    base = pl.multiple_of(wid * RW, RW)
    pltpu.sync_copy(wp_hbm.at[pl.ds(base, RW)], widx_v)
    pltpu.sync_copy(cp_hbm.at[pl.ds(base, RW)], pidx_v)

    lane = lax.iota(jnp.int32, NLANE)

    def idescr(c, ib, isem):
      cb = pl.multiple_of(c * C, C)
      return (np_hbm.at[pl.ds(base + cb, C), :], ib, isem)

    def descr(c, wr, pr, nr, ib, sem):
      cb = pl.multiple_of(c * C, C)
      return (
          (wt_hbm.at[widx_v.at[pl.ds(cb, C)]], wr, sem),
          (ct_hbm.at[pidx_v.at[pl.ds(cb, C)]], pr, sem),
          *[(ct_hbm.at[ib.at[i]],
             nr.at[pl.ds(i * NNEG, NNEG)], sem) for i in range(C)],
      )

    def issue(c, wr, pr, nr, ib, sem):
      for d in descr(c, wr, pr, nr, ib, sem):
        pltpu.async_copy(*d)

    def wait(c, wr, pr, nr, ib, sem):
      for d in descr(c, wr, pr, nr, ib, sem):
        pltpu.make_async_copy(*d).wait()

    def wbdescr(c, wb, wsem):
      cb = pl.multiple_of(c * C, C)
      return (wb, out_hbm.at[pl.ds((base + cb) * SROW, C * SROW)], wsem)

    def compute(c, wr, pr, nr, wb):
      cb = pl.multiple_of(c * C, C)

      def row_body(i, _):
        w = [wr[i, pl.ds(r * NLANE, NLANE)] for r in range(NREG)]

        def dot(crow):
          p = w[0] * crow[pl.ds(0, NLANE)]
          for r in range(1, NREG):
            p = p + w[r] * crow[pl.ds(r * NLANE, NLANE)]
          return _allsum(p)

        s_lo = jnp.where(lane == 0, dot(pr.at[i]), 0.0)
        s_hi = jnp.zeros((NLANE,), jnp.float32)
        for j in range(1, NCTX):
          tot = dot(nr.at[i * NNEG + (j - 1)])
          if j < NLANE:
            s_lo = jnp.where(lane == j, tot, s_lo)
          else:
            s_hi = jnp.where(lane == (j - NLANE), tot, s_hi)
        sb = i * SROW
        wb[pl.ds(sb, NLANE)] = s_lo
        wb[pl.ds(sb + NLANE, NLANE)] = s_hi
        return 0

      lax.fori_loop(0, C, row_body, 0)

    for s in range(GDIST):
      pltpu.sync_copy(*idescr(s, nix[s], isems[s])[:2])
    for y in range(GDIST, GDIST + IDIST):
      if y < NCHUNK:
        pltpu.async_copy(*idescr(y, nix[y % NIX], isems[y % NIX]))
    for s in range(GDIST):
      issue(s, wrs[s], prs[s], nrs[s], nix[s], sems[s])

    def chunk_body(c, _):
      for t in range(NIX):
        @pl.when(c % NIX == t)
        def _(t=t):
          yi = (t + GDIST + IDIST) % NIX

          @pl.when(c + GDIST + IDIST < NCHUNK)
          def _():
            pltpu.async_copy(
                *idescr(c + GDIST + IDIST, nix[yi], isems[yi]))

          gi = (t + GDIST) % NIX
          gs = (t + GDIST) % NSLOT

          @pl.when(c + GDIST < NCHUNK)
          def _():
            pltpu.make_async_copy(
                *idescr(c + GDIST, nix[gi], isems[gi])).wait()
            issue(c + GDIST, wrs[gs], prs[gs], nrs[gs], nix[gi], sems[gs])

          ts = t % NSLOT
          wait(c, wrs[ts], prs[ts], nrs[ts], nix[t], sems[ts])

          @pl.when(c >= NSLOT)
          def _():
            pltpu.make_async_copy(
                *wbdescr(c - NSLOT, wbs[ts], wsems[ts])).wait()
          compute(c, wrs[ts], prs[ts], nrs[ts], wbs[ts])
          pltpu.async_copy(*wbdescr(c, wbs[ts], wsems[ts]))

      return 0

    lax.fori_loop(0, NCHUNK, chunk_body, 0)
    for cc in range(NCHUNK - NSLOT, NCHUNK):
      pltpu.make_async_copy(
          *wbdescr(cc, wbs[cc % NSLOT], wsems[cc % NSLOT])).wait()

  return k(word_pos, ctx_pos, neg_pos, word_table, ctx_table)


TCR = B * SROW // 128
GRP = 128 // SROW


def _tc_finish(scores128):
  def body(s_ref, o_ref):
    x = s_ref[...]
    col = lax.broadcasted_iota(jnp.int32, (TCR, 128), 1)
    m = col % SROW
    val = jnp.where(m == 0, x, -x)
    ls = jnp.minimum(val, 0.0) - jnp.log1p(jnp.exp(-jnp.abs(val)))
    contrib = jnp.where(m <= NNEG, ls, 0.0)
    gi = lax.broadcasted_iota(jnp.int32, (128, GRP), 0) // SROW
    gj = lax.broadcasted_iota(jnp.int32, (128, GRP), 1)
    sel = jnp.where(gi == gj, -1.0, 0.0).astype(jnp.float32)
    o_ref[...] = jnp.dot(contrib, sel, preferred_element_type=jnp.float32,
                         precision=lax.Precision.HIGHEST)

  return pl.pallas_call(
      body,
      out_shape=jax.ShapeDtypeStruct((TCR, GRP), jnp.float32),
  )(scores128)


def kernel(word_pos, ctx_pos, neg_ctx_pos, word_table, ctx_table):
  word_pos = word_pos.astype(jnp.int32)
  ctx_pos = ctx_pos.astype(jnp.int32)
  neg_pos = neg_ctx_pos.astype(jnp.int32)
  scores = _sc_scores(word_pos, ctx_pos, neg_pos, word_table, ctx_table)
  return _tc_finish(scores.reshape(TCR, 128)).reshape(B)


# speedup vs baseline: 1.1819x; 1.1819x over previous
import functools

import jax
import jax.numpy as jnp
from jax import lax
from jax.experimental import pallas as pl
from jax.experimental.pallas import tpu as pltpu
from jax.experimental.pallas import tpu_sc as plsc

VOCAB = 100000
EMBED = 128
B = 16384
NNEG = 20
NCTX = NNEG + 1
NLANE = 16
NREG = EMBED // NLANE
SROW = 32

NC = 2
NS = 16
NW = NC * NS
RW = B // NW
C = 8
NCHUNK = RW // C
NI = C * NNEG
NSLOT = 4
NIX = 8
GDIST = NSLOT - 1
IDIST = NIX - NSLOT

_DNUMS = lax.GatherDimensionNumbers(
    offset_dims=(), collapsed_slice_dims=(0,), start_index_map=(0,))


def _lane_rot(p, sh):
  perm = ((lax.iota(jnp.int32, NLANE) + sh) % NLANE)[:, None]
  return lax.gather(p, perm, _DNUMS, (1,),
                    mode=lax.GatherScatterMode.PROMISE_IN_BOUNDS)


def _allsum(p):
  for sh in (8, 4, 2, 1):
    p = p + _lane_rot(p, sh)
  return p


def _sc_scores(word_pos, ctx_pos, neg_pos, word_table, ctx_table):
  mesh = plsc.VectorSubcoreMesh(core_axis_name="c", subcore_axis_name="s")

  @functools.partial(
      pl.kernel,
      mesh=mesh,
      out_type=jax.ShapeDtypeStruct((B * SROW,), jnp.float32),
      scratch_types=[
          pltpu.VMEM((RW,), jnp.int32),
          pltpu.VMEM((RW,), jnp.int32),
          *[pltpu.VMEM((C, NNEG), jnp.int32) for _ in range(NIX)],
          *[pltpu.VMEM((C, EMBED), jnp.float32) for _ in range(NSLOT)],
          *[pltpu.VMEM((C, EMBED), jnp.float32) for _ in range(NSLOT)],
          *[pltpu.VMEM((NI, EMBED), jnp.float32) for _ in range(NSLOT)],
          pltpu.VMEM((RW * SROW,), jnp.float32),
          *[pltpu.SemaphoreType.DMA for _ in range(NIX)],
          *[pltpu.SemaphoreType.DMA for _ in range(NSLOT)],
      ],
  )
  def k(wp_hbm, cp_hbm, np_hbm, wt_hbm, ct_hbm, out_hbm,
        widx_v, pidx_v, *rest):
    nix = rest[0:NIX]
    wrs = rest[NIX:NIX + NSLOT]
    prs = rest[NIX + NSLOT:NIX + 2 * NSLOT]
    nrs = rest[NIX + 2 * NSLOT:NIX + 3 * NSLOT]
    sbuf_v = rest[NIX + 3 * NSLOT]
    isems = rest[NIX + 3 * NSLOT + 1:NIX + 3 * NSLOT + 1 + NIX]
    sems = rest[NIX + 3 * NSLOT + 1 + NIX:]
    wid = lax.axis_index("s") * NC + lax.axis_index("c")
    base = pl.multiple_of(wid * RW, RW)
    pltpu.sync_copy(wp_hbm.at[pl.ds(base, RW)], widx_v)
    pltpu.sync_copy(cp_hbm.at[pl.ds(base, RW)], pidx_v)

    lane = lax.iota(jnp.int32, NLANE)

    def idescr(c, ib, isem):
      cb = pl.multiple_of(c * C, C)
      return (np_hbm.at[pl.ds(base + cb, C), :], ib, isem)

    def descr(c, wr, pr, nr, ib, sem):
      cb = pl.multiple_of(c * C, C)
      return (
          (wt_hbm.at[widx_v.at[pl.ds(cb, C)]], wr, sem),
          (ct_hbm.at[pidx_v.at[pl.ds(cb, C)]], pr, sem),
          *[(ct_hbm.at[ib.at[i]],
             nr.at[pl.ds(i * NNEG, NNEG)], sem) for i in range(C)],
      )

    def issue(c, wr, pr, nr, ib, sem):
      for d in descr(c, wr, pr, nr, ib, sem):
        pltpu.async_copy(*d)

    def wait(c, wr, pr, nr, ib, sem):
      for d in descr(c, wr, pr, nr, ib, sem):
        pltpu.make_async_copy(*d).wait()

    def compute(c, wr, pr, nr):
      cb = pl.multiple_of(c * C, C)

      def row_body(i, _):
        w = [wr[i, pl.ds(r * NLANE, NLANE)] for r in range(NREG)]

        def dot(crow):
          p = w[0] * crow[pl.ds(0, NLANE)]
          for r in range(1, NREG):
            p = p + w[r] * crow[pl.ds(r * NLANE, NLANE)]
          return _allsum(p)

        s_lo = jnp.where(lane == 0, dot(pr.at[i]), 0.0)
        s_hi = jnp.zeros((NLANE,), jnp.float32)
        for j in range(1, NCTX):
          tot = dot(nr.at[i * NNEG + (j - 1)])
          if j < NLANE:
            s_lo = jnp.where(lane == j, tot, s_lo)
          else:
            s_hi = jnp.where(lane == (j - NLANE), tot, s_hi)
        sb = (cb + i) * SROW
        sbuf_v[pl.ds(sb, NLANE)] = s_lo
        sbuf_v[pl.ds(sb + NLANE, NLANE)] = s_hi
        return 0

      lax.fori_loop(0, C, row_body, 0)

    for s in range(GDIST):
      pltpu.sync_copy(*idescr(s, nix[s], isems[s])[:2])
    for y in range(GDIST, GDIST + IDIST):
      if y < NCHUNK:
        pltpu.async_copy(*idescr(y, nix[y % NIX], isems[y % NIX]))
    for s in range(GDIST):
      issue(s, wrs[s], prs[s], nrs[s], nix[s], sems[s])

    def chunk_body(c, _):
      for t in range(NIX):
        @pl.when(c % NIX == t)
        def _(t=t):
          yi = (t + GDIST + IDIST) % NIX

          @pl.when(c + GDIST + IDIST < NCHUNK)
          def _():
            pltpu.async_copy(
                *idescr(c + GDIST + IDIST, nix[yi], isems[yi]))

          gi = (t + GDIST) % NIX
          gs = (t + GDIST) % NSLOT

          @pl.when(c + GDIST < NCHUNK)
          def _():
            pltpu.make_async_copy(
                *idescr(c + GDIST, nix[gi], isems[gi])).wait()
            issue(c + GDIST, wrs[gs], prs[gs], nrs[gs], nix[gi], sems[gs])

          ts = t % NSLOT
          wait(c, wrs[ts], prs[ts], nrs[ts], nix[t], sems[ts])

      for s in range(NSLOT):
        @pl.when(c % NSLOT == s)
        def _(s=s):
          compute(c, wrs[s], prs[s], nrs[s])

      return 0

    lax.fori_loop(0, NCHUNK, chunk_body, 0)
    pltpu.sync_copy(sbuf_v, out_hbm.at[pl.ds(base * SROW, RW * SROW)])

  return k(word_pos, ctx_pos, neg_pos, word_table, ctx_table)


TCR = B * SROW // 128
GRP = 128 // SROW


def _tc_finish(scores128):
  def body(s_ref, o_ref):
    x = s_ref[...]
    col = lax.broadcasted_iota(jnp.int32, (TCR, 128), 1)
    m = col % SROW
    val = jnp.where(m == 0, x, -x)
    ls = jnp.minimum(val, 0.0) - jnp.log1p(jnp.exp(-jnp.abs(val)))
    contrib = jnp.where(m <= NNEG, ls, 0.0)
    gi = lax.broadcasted_iota(jnp.int32, (128, GRP), 0) // SROW
    gj = lax.broadcasted_iota(jnp.int32, (128, GRP), 1)
    sel = jnp.where(gi == gj, -1.0, 0.0).astype(jnp.float32)
    o_ref[...] = jnp.dot(contrib, sel, preferred_element_type=jnp.float32,
                         precision=lax.Precision.HIGHEST)

  return pl.pallas_call(
      body,
      out_shape=jax.ShapeDtypeStruct((TCR, GRP), jnp.float32),
  )(scores128)


def kernel(word_pos, ctx_pos, neg_ctx_pos, word_table, ctx_table):
  word_pos = word_pos.astype(jnp.int32)
  ctx_pos = ctx_pos.astype(jnp.int32)
  neg_pos = neg_ctx_pos.astype(jnp.int32)
  scores = _sc_scores(word_pos, ctx_pos, neg_pos, word_table, ctx_table)
  return _tc_finish(scores.reshape(TCR, 128)).reshape(B)
